# Initial kernel scaffold; baseline (speedup 1.0000x reference)
import jax
import jax.numpy as jnp
from jax.experimental import pallas as pl


def kernel(prob, mat_rows, mat_cols, mat_vals, num_edges):
    raise NotImplementedError("write your pallas kernel here")


# speedup vs baseline: 19.0113x; 19.0113x over previous
import functools

import jax
import jax.numpy as jnp
from jax import lax
from jax.experimental import pallas as pl
from jax.experimental.pallas import tpu as pltpu
from jax.experimental.pallas import tpu_sc as plsc

_NC = 2
_NS = 16
_NW = _NC * _NS
_L = 16
_IDX = 128
_SUB = 4
_BLK = _SUB * _IDX
_UNROLL = 16


@functools.lru_cache(maxsize=None)
def _make_sc_kernel(n: int, k: int, nb: int, interpret: bool = False):
    assert k % _L == 0
    kh = k // _L
    mesh = plsc.VectorSubcoreMesh(
        core_axis_name="c", subcore_axis_name="s",
        num_cores=_NC, num_subcores=_NS)

    @functools.partial(
        pl.kernel,
        out_type=jax.ShapeDtypeStruct((_NW, _L), jnp.float32),
        mesh=mesh,
        scratch_types=[
            pltpu.VMEM((_SUB, _IDX), jnp.int32),
            pltpu.VMEM((_SUB, _IDX), jnp.int32),
            pltpu.VMEM((_SUB, _IDX), jnp.float32),
            pltpu.VMEM((_SUB, _IDX, k), jnp.float32),
            pltpu.VMEM((_SUB, _IDX, k), jnp.float32),
            pltpu.VMEM((_L,), jnp.float32),
            pltpu.SemaphoreType.DMA,
        ],
        compiler_params=pltpu.CompilerParams(use_tc_tiling_on_sc=False),
        interpret=interpret,
    )
    def sc_kernel(prob_hbm, rows_hbm, cols_hbm, vals_hbm, out_hbm,
                  rows_v, cols_v, vals_v, a_v, b_v, acc_v, sem):
        wid = lax.axis_index("s") * _NC + lax.axis_index("c")

        def block_body(i, acc):
            r0 = (wid * nb + i) * _SUB
            pltpu.sync_copy(rows_hbm.at[pl.ds(r0, _SUB)], rows_v)
            pltpu.sync_copy(cols_hbm.at[pl.ds(r0, _SUB)], cols_v)
            pltpu.sync_copy(vals_hbm.at[pl.ds(r0, _SUB)], vals_v)
            copies = []
            for j in range(_SUB):
                copies.append(pltpu.async_copy(
                    prob_hbm.at[rows_v.at[j]], a_v.at[j], sem))
                copies.append(pltpu.async_copy(
                    prob_hbm.at[cols_v.at[j]], b_v.at[j], sem))
            for cp in copies:
                cp.wait()

            for j in range(_SUB):
                def edge_body(t, acc, j=j):
                    e0 = t * _UNROLL
                    vv = vals_v[j, pl.ds(e0, _UNROLL)]
                    for u in range(_UNROLL):
                        e = e0 + u
                        dot = None
                        for h in range(kh):
                            p = (a_v[j, e, pl.ds(h * _L, _L)]
                                 * b_v[j, e, pl.ds(h * _L, _L)])
                            dot = p if dot is None else dot + p
                        acc = acc + vv[u] * dot
                    return acc
                acc = lax.fori_loop(0, _IDX // _UNROLL, edge_body, acc)
            return acc

        acc = lax.fori_loop(0, nb, block_body, jnp.zeros((_L,), jnp.float32))
        acc_v[...] = acc
        pltpu.sync_copy(acc_v, out_hbm.at[wid])

    return sc_kernel


def kernel(prob, mat_rows, mat_cols, mat_vals, num_edges):
    n, k = prob.shape
    e = mat_rows.shape[0]
    nb = -(-e // (_NW * _BLK))
    e_pad = _NW * nb * _BLK
    pad = e_pad - e
    rows = jnp.pad(mat_rows.astype(jnp.int32), (0, pad)).reshape(-1, _IDX)
    cols = jnp.pad(mat_cols.astype(jnp.int32), (0, pad)).reshape(-1, _IDX)
    vals = jnp.pad(mat_vals, (0, pad)).reshape(-1, _IDX)
    partials = _make_sc_kernel(n, k, nb)(prob, rows, cols, vals)
    return jnp.reshape(jnp.sum(partials), (1,)) / num_edges


# speedup vs baseline: 22.7461x; 1.1965x over previous
import functools

import jax
import jax.numpy as jnp
from jax import lax
from jax.experimental import pallas as pl
from jax.experimental.pallas import tpu as pltpu
from jax.experimental.pallas import tpu_sc as plsc

_NC = 2
_NS = 16
_NW = _NC * _NS
_L = 16
_IDX = 128
_SUB = 4
_BLK = _SUB * _IDX
_UNROLL = 16
_NACC = 4


@functools.lru_cache(maxsize=None)
def _make_sc_kernel(n: int, k: int, nb: int):
    assert k % _L == 0
    kh = k // _L
    mesh = plsc.VectorSubcoreMesh(
        core_axis_name="c", subcore_axis_name="s",
        num_cores=_NC, num_subcores=_NS)

    @functools.partial(
        pl.kernel,
        out_type=jax.ShapeDtypeStruct((_NW, _L), jnp.float32),
        mesh=mesh,
        scratch_types=[
            pltpu.VMEM((2, _SUB, _IDX), jnp.int32),
            pltpu.VMEM((2, _SUB, _IDX), jnp.int32),
            pltpu.VMEM((2, _SUB, _IDX), jnp.float32),
            pltpu.VMEM((2, _SUB, _IDX, k), jnp.float32),
            pltpu.VMEM((2, _SUB, _IDX, k), jnp.float32),
            pltpu.VMEM((_L,), jnp.float32),
            pltpu.SemaphoreType.DMA,
        ],
        compiler_params=pltpu.CompilerParams(use_tc_tiling_on_sc=False),
    )
    def sc_kernel(prob_hbm, rows_hbm, cols_hbm, vals_hbm, out_hbm,
                  rows_v, cols_v, vals_v, a_v, b_v, acc_v, sem):
        wid = lax.axis_index("s") * _NC + lax.axis_index("c")

        def load_and_gather(i, buf):
            r0 = (wid * nb + i) * _SUB
            pltpu.sync_copy(rows_hbm.at[pl.ds(r0, _SUB)], rows_v.at[buf])
            pltpu.sync_copy(cols_hbm.at[pl.ds(r0, _SUB)], cols_v.at[buf])
            pltpu.sync_copy(vals_hbm.at[pl.ds(r0, _SUB)], vals_v.at[buf])
            for j in range(_SUB):
                pltpu.async_copy(
                    prob_hbm.at[rows_v.at[buf, j]], a_v.at[buf, j], sem)
                pltpu.async_copy(
                    prob_hbm.at[cols_v.at[buf, j]], b_v.at[buf, j], sem)

        def wait_gathers(buf):
            for j in range(_SUB):
                pltpu.make_async_copy(
                    prob_hbm.at[rows_v.at[buf, j]], a_v.at[buf, j], sem).wait()
                pltpu.make_async_copy(
                    prob_hbm.at[cols_v.at[buf, j]], b_v.at[buf, j], sem).wait()

        load_and_gather(0, 0)

        def block_body(i, accs):
            buf = lax.rem(i, 2)
            wait_gathers(buf)
            load_and_gather(lax.rem(i + 1, nb), 1 - buf)

            for j in range(_SUB):
                def edge_body(t, accs, j=j):
                    accs = list(accs)
                    e0 = t * _UNROLL
                    vv = vals_v[buf, j, pl.ds(e0, _UNROLL)]
                    for u in range(_UNROLL):
                        e = e0 + u
                        dot = None
                        for h in range(kh):
                            p = (a_v[buf, j, e, pl.ds(h * _L, _L)]
                                 * b_v[buf, j, e, pl.ds(h * _L, _L)])
                            dot = p if dot is None else dot + p
                        accs[u % _NACC] = accs[u % _NACC] + vv[u] * dot
                    return tuple(accs)
                accs = lax.fori_loop(0, _IDX // _UNROLL, edge_body, accs)
            return accs

        zeros = jnp.zeros((_L,), jnp.float32)
        accs = lax.fori_loop(0, nb, block_body, (zeros,) * _NACC)
        wait_gathers(nb % 2)
        acc = accs[0]
        for a in accs[1:]:
            acc = acc + a
        acc_v[...] = acc
        pltpu.sync_copy(acc_v, out_hbm.at[wid])

    return sc_kernel


def kernel(prob, mat_rows, mat_cols, mat_vals, num_edges):
    n, k = prob.shape
    e = mat_rows.shape[0]
    nb = -(-e // (_NW * _BLK))
    e_pad = _NW * nb * _BLK
    pad = e_pad - e
    rows = jnp.pad(mat_rows.astype(jnp.int32), (0, pad)).reshape(-1, _IDX)
    cols = jnp.pad(mat_cols.astype(jnp.int32), (0, pad)).reshape(-1, _IDX)
    vals = jnp.pad(mat_vals, (0, pad)).reshape(-1, _IDX)
    partials = _make_sc_kernel(n, k, nb)(prob, rows, cols, vals)
    return jnp.reshape(jnp.sum(partials), (1,)) / num_edges


# speedup vs baseline: 24.6949x; 1.0857x over previous
import functools

import jax
import jax.numpy as jnp
from jax import lax
from jax.experimental import pallas as pl
from jax.experimental.pallas import tpu as pltpu
from jax.experimental.pallas import tpu_sc as plsc

_NC = 2
_NS = 16
_NW = _NC * _NS
_L = 16
_IDX = 128
_SUB = 4
_BLK = _SUB * _IDX
_UNROLL = 16
_NACC = 4


@functools.lru_cache(maxsize=None)
def _make_sc_kernel(n: int, k: int, nb: int):
    assert k % (2 * _L) == 0
    kh = k // (2 * _L)
    mesh = plsc.VectorSubcoreMesh(
        core_axis_name="c", subcore_axis_name="s",
        num_cores=_NC, num_subcores=_NS)

    @functools.partial(
        pl.kernel,
        out_type=jax.ShapeDtypeStruct((_NW, _L), jnp.float32),
        mesh=mesh,
        scratch_types=[
            pltpu.VMEM((2, _SUB, _IDX), jnp.int32),
            pltpu.VMEM((2, _SUB, _IDX), jnp.int32),
            pltpu.VMEM((2, _SUB, _IDX), jnp.float32),
            pltpu.VMEM((2, _SUB, _IDX, k), jnp.bfloat16),
            pltpu.VMEM((2, _SUB, _IDX, k), jnp.bfloat16),
            pltpu.VMEM((_L,), jnp.float32),
            pltpu.SemaphoreType.DMA,
        ],
        compiler_params=pltpu.CompilerParams(use_tc_tiling_on_sc=False, needs_layout_passes=False),
    )
    def sc_kernel(prob_hbm, rows_hbm, cols_hbm, vals_hbm, out_hbm,
                  rows_v, cols_v, vals_v, a_v, b_v, acc_v, sem):
        wid = lax.axis_index("s") * _NC + lax.axis_index("c")

        def load_and_gather(i, buf):
            r0 = (wid * nb + i) * _SUB
            pltpu.sync_copy(rows_hbm.at[pl.ds(r0, _SUB)], rows_v.at[buf])
            pltpu.sync_copy(cols_hbm.at[pl.ds(r0, _SUB)], cols_v.at[buf])
            pltpu.sync_copy(vals_hbm.at[pl.ds(r0, _SUB)], vals_v.at[buf])
            for j in range(_SUB):
                pltpu.async_copy(
                    prob_hbm.at[rows_v.at[buf, j]], a_v.at[buf, j], sem)
                pltpu.async_copy(
                    prob_hbm.at[cols_v.at[buf, j]], b_v.at[buf, j], sem)

        def wait_gathers(buf):
            for j in range(_SUB):
                pltpu.make_async_copy(
                    prob_hbm.at[rows_v.at[buf, j]], a_v.at[buf, j], sem).wait()
                pltpu.make_async_copy(
                    prob_hbm.at[cols_v.at[buf, j]], b_v.at[buf, j], sem).wait()

        load_and_gather(0, 0)

        def block_body(i, accs):
            buf = lax.rem(i, 2)
            wait_gathers(buf)
            load_and_gather(lax.rem(i + 1, nb), 1 - buf)

            for j in range(_SUB):
                def edge_body(t, accs, j=j):
                    accs = list(accs)
                    e0 = t * _UNROLL
                    vv = vals_v[buf, j, pl.ds(e0, _UNROLL)]
                    for u in range(_UNROLL):
                        e = e0 + u
                        dot = None
                        for h in range(kh):
                            p = (a_v[buf, j, e, pl.ds(h * 2 * _L, 2 * _L)]
                                 * b_v[buf, j, e, pl.ds(h * 2 * _L, 2 * _L)])
                            p0, p1 = plsc.unpack(
                                p, format=plsc.PackFormat.INTERLEAVED)
                            s = p0 + p1
                            dot = s if dot is None else dot + s
                        accs[u % _NACC] = accs[u % _NACC] + vv[u] * dot
                    return tuple(accs)
                accs = lax.fori_loop(0, _IDX // _UNROLL, edge_body, accs)
            return accs

        zeros = jnp.zeros((_L,), jnp.float32)
        accs = lax.fori_loop(0, nb, block_body, (zeros,) * _NACC)
        wait_gathers(nb % 2)
        acc = accs[0]
        for a in accs[1:]:
            acc = acc + a
        acc_v[...] = acc
        pltpu.sync_copy(acc_v, out_hbm.at[wid])

    return sc_kernel


def kernel(prob, mat_rows, mat_cols, mat_vals, num_edges):
    n, k = prob.shape
    e = mat_rows.shape[0]
    nb = -(-e // (_NW * _BLK))
    e_pad = _NW * nb * _BLK
    pad = e_pad - e
    rows = jnp.pad(mat_rows.astype(jnp.int32), (0, pad)).reshape(-1, _IDX)
    cols = jnp.pad(mat_cols.astype(jnp.int32), (0, pad)).reshape(-1, _IDX)
    vals = jnp.pad(mat_vals, (0, pad)).reshape(-1, _IDX)
    partials = _make_sc_kernel(n, k, nb)(
        prob.astype(jnp.bfloat16), rows, cols, vals)
    return jnp.reshape(jnp.sum(partials), (1,)) / num_edges


# speedup vs baseline: 36.4445x; 1.4758x over previous
import functools

import jax
import jax.numpy as jnp
from jax import lax
from jax.experimental import pallas as pl
from jax.experimental.pallas import tpu as pltpu
from jax.experimental.pallas import tpu_sc as plsc

_NC = 2
_NS = 16
_NW = _NC * _NS
_L = 16
_IDX = 128
_SUB = 4
_BLK = _SUB * _IDX
_UNROLL = 16
_NACC = 4


@functools.lru_cache(maxsize=None)
def _make_sc_kernel(n: int, k: int, nb: int):
    assert k % (2 * _L) == 0
    kh = k // (2 * _L)
    rows_per_tile = nb * _SUB
    mesh = plsc.VectorSubcoreMesh(
        core_axis_name="c", subcore_axis_name="s",
        num_cores=_NC, num_subcores=_NS)

    @functools.partial(
        pl.kernel,
        out_type=jax.ShapeDtypeStruct((_NW, _L), jnp.float32),
        mesh=mesh,
        scratch_types=[
            pltpu.VMEM((rows_per_tile, _IDX), jnp.int32),
            pltpu.VMEM((rows_per_tile, _IDX), jnp.int32),
            pltpu.VMEM((rows_per_tile, _IDX), jnp.float32),
            pltpu.VMEM((2, _SUB, _IDX, k), jnp.bfloat16),
            pltpu.VMEM((2, _SUB, _IDX, k), jnp.bfloat16),
            pltpu.VMEM((_L,), jnp.float32),
            pltpu.SemaphoreType.DMA,
        ],
        compiler_params=pltpu.CompilerParams(
            use_tc_tiling_on_sc=False, needs_layout_passes=False),
    )
    def sc_kernel(prob_hbm, rows_hbm, cols_hbm, vals_hbm, out_hbm,
                  rows_v, cols_v, vals_v, a_v, b_v, acc_v, sem):
        wid = lax.axis_index("s") * _NC + lax.axis_index("c")
        base = wid * rows_per_tile
        pltpu.sync_copy(rows_hbm.at[pl.ds(base, rows_per_tile)], rows_v)
        pltpu.sync_copy(cols_hbm.at[pl.ds(base, rows_per_tile)], cols_v)
        pltpu.sync_copy(vals_hbm.at[pl.ds(base, rows_per_tile)], vals_v)

        def start_gathers(i, buf):
            for j in range(_SUB):
                pltpu.async_copy(
                    prob_hbm.at[rows_v.at[i * _SUB + j]], a_v.at[buf, j], sem)
                pltpu.async_copy(
                    prob_hbm.at[cols_v.at[i * _SUB + j]], b_v.at[buf, j], sem)

        def wait_gathers(i, buf):
            for j in range(_SUB):
                pltpu.make_async_copy(
                    prob_hbm.at[rows_v.at[i * _SUB + j]], a_v.at[buf, j],
                    sem).wait()
                pltpu.make_async_copy(
                    prob_hbm.at[cols_v.at[i * _SUB + j]], b_v.at[buf, j],
                    sem).wait()

        start_gathers(0, 0)

        def block_body(i, accs):
            buf = lax.rem(i, 2)
            start_gathers(lax.rem(i + 1, nb), 1 - buf)
            wait_gathers(i, buf)

            for j in range(_SUB):
                def edge_body(t, accs, j=j):
                    accs = list(accs)
                    e0 = t * _UNROLL
                    vv = vals_v[i * _SUB + j, pl.ds(e0, _UNROLL)]
                    for u in range(_UNROLL):
                        e = e0 + u
                        dot = None
                        for h in range(kh):
                            p = (a_v[buf, j, e, pl.ds(h * 2 * _L, 2 * _L)]
                                 * b_v[buf, j, e, pl.ds(h * 2 * _L, 2 * _L)])
                            p0, p1 = plsc.unpack(
                                p, format=plsc.PackFormat.INTERLEAVED)
                            s = p0 + p1
                            dot = s if dot is None else dot + s
                        accs[u % _NACC] = accs[u % _NACC] + vv[u] * dot
                    return tuple(accs)
                accs = lax.fori_loop(0, _IDX // _UNROLL, edge_body, accs)
            return accs

        zeros = jnp.zeros((_L,), jnp.float32)
        accs = lax.fori_loop(0, nb, block_body, (zeros,) * _NACC)
        wait_gathers(0, nb % 2)
        acc = accs[0]
        for a in accs[1:]:
            acc = acc + a
        acc_v[...] = acc
        pltpu.sync_copy(acc_v, out_hbm.at[wid])

    return sc_kernel


def kernel(prob, mat_rows, mat_cols, mat_vals, num_edges):
    n, k = prob.shape
    e = mat_rows.shape[0]
    nb = -(-e // (_NW * _BLK))
    e_pad = _NW * nb * _BLK
    pad = e_pad - e
    rows = jnp.pad(mat_rows.astype(jnp.int32), (0, pad)).reshape(-1, _IDX)
    cols = jnp.pad(mat_cols.astype(jnp.int32), (0, pad)).reshape(-1, _IDX)
    vals = jnp.pad(mat_vals, (0, pad)).reshape(-1, _IDX)
    partials = _make_sc_kernel(n, k, nb)(
        prob.astype(jnp.bfloat16), rows, cols, vals)
    return jnp.reshape(jnp.sum(partials), (1,)) / num_edges


# speedup vs baseline: 37.2687x; 1.0226x over previous
import functools

import jax
import jax.numpy as jnp
from jax import lax
from jax.experimental import pallas as pl
from jax.experimental.pallas import tpu as pltpu
from jax.experimental.pallas import tpu_sc as plsc

_NC = 2
_NS = 16
_NW = _NC * _NS
_L = 16
_IDX = 128
_SUB = 4
_BLK = _SUB * _IDX
_UNROLL = 16
_NACC = 4


@functools.lru_cache(maxsize=None)
def _make_sc_kernel(n: int, k: int, nb: int):
    assert k % (2 * _L) == 0
    kh = k // (2 * _L)
    mesh = plsc.VectorSubcoreMesh(
        core_axis_name="c", subcore_axis_name="s",
        num_cores=_NC, num_subcores=_NS)

    @functools.partial(
        pl.kernel,
        out_type=jax.ShapeDtypeStruct((_NW, _L), jnp.float32),
        mesh=mesh,
        scratch_types=[
            pltpu.VMEM((3, _SUB, _IDX), jnp.int32),
            pltpu.VMEM((3, _SUB, _IDX), jnp.int32),
            pltpu.VMEM((3, _SUB, _IDX), jnp.float32),
            pltpu.VMEM((2, _SUB, _IDX, k), jnp.bfloat16),
            pltpu.VMEM((2, _SUB, _IDX, k), jnp.bfloat16),
            pltpu.VMEM((_L,), jnp.float32),
            pltpu.SemaphoreType.DMA,
            pltpu.SemaphoreType.DMA,
        ],
        compiler_params=pltpu.CompilerParams(
            use_tc_tiling_on_sc=False, needs_layout_passes=False),
    )
    def sc_kernel(prob_hbm, rows_hbm, cols_hbm, vals_hbm, out_hbm,
                  rows_v, cols_v, vals_v, a_v, b_v, acc_v,
                  sem_i, sem_g):
        wid = lax.axis_index("s") * _NC + lax.axis_index("c")
        base = wid * nb * _SUB

        def issue_idx(i):
            ib = lax.rem(i, 3)
            r0 = base + lax.rem(i, nb) * _SUB
            pltpu.async_copy(rows_hbm.at[pl.ds(r0, _SUB)], rows_v.at[ib],
                             sem_i)
            pltpu.async_copy(cols_hbm.at[pl.ds(r0, _SUB)], cols_v.at[ib],
                             sem_i)
            pltpu.async_copy(vals_hbm.at[pl.ds(r0, _SUB)], vals_v.at[ib],
                             sem_i)

        def wait_idx():
            pltpu.make_async_copy(rows_hbm.at[pl.ds(base, _SUB)],
                                  rows_v.at[0], sem_i).wait()
            pltpu.make_async_copy(cols_hbm.at[pl.ds(base, _SUB)],
                                  cols_v.at[0], sem_i).wait()
            pltpu.make_async_copy(vals_hbm.at[pl.ds(base, _SUB)],
                                  vals_v.at[0], sem_i).wait()

        def issue_gathers(i):
            ib = lax.rem(i, 3)
            gb = lax.rem(i, 2)
            for j in range(_SUB):
                pltpu.async_copy(
                    prob_hbm.at[rows_v.at[ib, j]], a_v.at[gb, j], sem_g)
                pltpu.async_copy(
                    prob_hbm.at[cols_v.at[ib, j]], b_v.at[gb, j], sem_g)

        def wait_gathers():
            for j in range(_SUB):
                pltpu.make_async_copy(
                    prob_hbm.at[rows_v.at[0, j]], a_v.at[0, j], sem_g).wait()
                pltpu.make_async_copy(
                    prob_hbm.at[cols_v.at[0, j]], b_v.at[0, j], sem_g).wait()

        issue_idx(0)
        issue_idx(1)
        wait_idx()
        issue_gathers(0)

        def block_body(i, accs):
            issue_idx(i + 2)
            wait_idx()
            issue_gathers(i + 1)
            wait_gathers()

            ib = lax.rem(i, 3)
            gb = lax.rem(i, 2)
            for j in range(_SUB):
                def edge_body(t, accs, j=j):
                    accs = list(accs)
                    e0 = t * _UNROLL
                    vv = vals_v[ib, j, pl.ds(e0, _UNROLL)]
                    for u in range(_UNROLL):
                        e = e0 + u
                        dot = None
                        for h in range(kh):
                            p = (a_v[gb, j, e, pl.ds(h * 2 * _L, 2 * _L)]
                                 * b_v[gb, j, e, pl.ds(h * 2 * _L, 2 * _L)])
                            p0, p1 = plsc.unpack(
                                p, format=plsc.PackFormat.INTERLEAVED)
                            s = p0 + p1
                            dot = s if dot is None else dot + s
                        accs[u % _NACC] = accs[u % _NACC] + vv[u] * dot
                    return tuple(accs)
                accs = lax.fori_loop(0, _IDX // _UNROLL, edge_body, accs)
            return accs

        zeros = jnp.zeros((_L,), jnp.float32)
        accs = lax.fori_loop(0, nb, block_body, (zeros,) * _NACC)
        wait_idx()
        wait_gathers()
        acc = accs[0]
        for a in accs[1:]:
            acc = acc + a
        acc_v[...] = acc
        pltpu.sync_copy(acc_v, out_hbm.at[wid])

    return sc_kernel


def kernel(prob, mat_rows, mat_cols, mat_vals, num_edges):
    n, k = prob.shape
    e = mat_rows.shape[0]
    nb = -(-e // (_NW * _BLK))
    e_pad = _NW * nb * _BLK
    pad = e_pad - e
    rows = jnp.pad(mat_rows.astype(jnp.int32), (0, pad)).reshape(-1, _IDX)
    cols = jnp.pad(mat_cols.astype(jnp.int32), (0, pad)).reshape(-1, _IDX)
    vals = jnp.pad(mat_vals, (0, pad)).reshape(-1, _IDX)
    partials = _make_sc_kernel(n, k, nb)(
        prob.astype(jnp.bfloat16), rows, cols, vals)
    return jnp.reshape(jnp.sum(partials), (1,)) / num_edges


# speedup vs baseline: 55.6768x; 1.4939x over previous
import functools

import jax
import jax.numpy as jnp
from jax import lax
from jax.experimental import pallas as pl
from jax.experimental.pallas import tpu as pltpu
from jax.experimental.pallas import tpu_sc as plsc

_NC = 2
_NS = 16
_NW = _NC * _NS
_L = 16
_IDX = 128
_SUB = 4
_BLK = _SUB * _IDX
_UNROLL = 16
_NACC = 4


@functools.lru_cache(maxsize=None)
def _make_sc_kernel(n: int, k: int, nb: int):
    assert k % (2 * _L) == 0
    kh = k // (2 * _L)
    mesh = plsc.VectorSubcoreMesh(
        core_axis_name="c", subcore_axis_name="s",
        num_cores=_NC, num_subcores=_NS)

    @functools.partial(
        pl.kernel,
        out_type=jax.ShapeDtypeStruct((_NW, _L), jnp.float32),
        mesh=mesh,
        scratch_types=[
            pltpu.VMEM((3, _SUB, _IDX), jnp.int32),
            pltpu.VMEM((3, _SUB, _IDX), jnp.int32),
            pltpu.VMEM((3, _SUB, _IDX), jnp.float32),
            pltpu.VMEM_SHARED((n, k), jnp.bfloat16),
            pltpu.VMEM((2, _SUB, _IDX, k), jnp.bfloat16),
            pltpu.VMEM((2, _SUB, _IDX, k), jnp.bfloat16),
            pltpu.VMEM((_L,), jnp.float32),
            pltpu.SemaphoreType.DMA,
            pltpu.SemaphoreType.DMA,
        ],
        compiler_params=pltpu.CompilerParams(
            use_tc_tiling_on_sc=False, needs_layout_passes=False),
    )
    def sc_kernel(prob_hbm, rows_hbm, cols_hbm, vals_hbm, out_hbm,
                  rows_v, cols_v, vals_v, prob_sp, a_v, b_v, acc_v,
                  sem_i, sem_g):
        wid = lax.axis_index("s") * _NC + lax.axis_index("c")
        sid = lax.axis_index("s")
        base = wid * nb * _SUB
        rows_per_sub = n // _NS
        pltpu.sync_copy(prob_hbm.at[pl.ds(sid * rows_per_sub, rows_per_sub)],
                        prob_sp.at[pl.ds(sid * rows_per_sub, rows_per_sub)])

        def issue_idx(i):
            ib = lax.rem(i, 3)
            r0 = base + lax.rem(i, nb) * _SUB
            pltpu.async_copy(rows_hbm.at[pl.ds(r0, _SUB)], rows_v.at[ib],
                             sem_i)
            pltpu.async_copy(cols_hbm.at[pl.ds(r0, _SUB)], cols_v.at[ib],
                             sem_i)
            pltpu.async_copy(vals_hbm.at[pl.ds(r0, _SUB)], vals_v.at[ib],
                             sem_i)

        def wait_idx():
            pltpu.make_async_copy(rows_hbm.at[pl.ds(base, _SUB)],
                                  rows_v.at[0], sem_i).wait()
            pltpu.make_async_copy(cols_hbm.at[pl.ds(base, _SUB)],
                                  cols_v.at[0], sem_i).wait()
            pltpu.make_async_copy(vals_hbm.at[pl.ds(base, _SUB)],
                                  vals_v.at[0], sem_i).wait()

        def issue_gathers(i):
            ib = lax.rem(i, 3)
            gb = lax.rem(i, 2)
            for j in range(_SUB):
                pltpu.async_copy(
                    prob_sp.at[rows_v.at[ib, j]], a_v.at[gb, j], sem_g)
                pltpu.async_copy(
                    prob_sp.at[cols_v.at[ib, j]], b_v.at[gb, j], sem_g)

        def wait_gathers():
            for j in range(_SUB):
                pltpu.make_async_copy(
                    prob_sp.at[rows_v.at[0, j]], a_v.at[0, j], sem_g).wait()
                pltpu.make_async_copy(
                    prob_sp.at[cols_v.at[0, j]], b_v.at[0, j], sem_g).wait()

        issue_idx(0)
        issue_idx(1)
        plsc.subcore_barrier()
        wait_idx()
        issue_gathers(0)

        def block_body(i, accs):
            issue_idx(i + 2)
            wait_idx()
            issue_gathers(i + 1)
            wait_gathers()

            ib = lax.rem(i, 3)
            gb = lax.rem(i, 2)
            for j in range(_SUB):
                def edge_body(t, accs, j=j):
                    accs = list(accs)
                    e0 = t * _UNROLL
                    vv = vals_v[ib, j, pl.ds(e0, _UNROLL)]
                    for u in range(_UNROLL):
                        e = e0 + u
                        dot = None
                        for h in range(kh):
                            p = (a_v[gb, j, e, pl.ds(h * 2 * _L, 2 * _L)]
                                 * b_v[gb, j, e, pl.ds(h * 2 * _L, 2 * _L)])
                            p0, p1 = plsc.unpack(
                                p, format=plsc.PackFormat.INTERLEAVED)
                            s = p0 + p1
                            dot = s if dot is None else dot + s
                        accs[u % _NACC] = accs[u % _NACC] + vv[u] * dot
                    return tuple(accs)
                accs = lax.fori_loop(0, _IDX // _UNROLL, edge_body, accs)
            return accs

        zeros = jnp.zeros((_L,), jnp.float32)
        accs = lax.fori_loop(0, nb, block_body, (zeros,) * _NACC)
        wait_idx()
        wait_gathers()
        acc = accs[0]
        for a in accs[1:]:
            acc = acc + a
        acc_v[...] = acc
        pltpu.sync_copy(acc_v, out_hbm.at[wid])

    return sc_kernel


def kernel(prob, mat_rows, mat_cols, mat_vals, num_edges):
    n, k = prob.shape
    e = mat_rows.shape[0]
    nb = -(-e // (_NW * _BLK))
    e_pad = _NW * nb * _BLK
    pad = e_pad - e
    rows = jnp.pad(mat_rows.astype(jnp.int32), (0, pad)).reshape(-1, _IDX)
    cols = jnp.pad(mat_cols.astype(jnp.int32), (0, pad)).reshape(-1, _IDX)
    vals = jnp.pad(mat_vals, (0, pad)).reshape(-1, _IDX)
    partials = _make_sc_kernel(n, k, nb)(
        prob.astype(jnp.bfloat16), rows, cols, vals)
    return jnp.reshape(jnp.sum(partials), (1,)) / num_edges
